# baseline (speedup 1.0000x reference)
import functools

import jax
import jax.numpy as jnp
from jax import lax
from jax.experimental import pallas as pl
from jax.experimental.pallas import tpu as pltpu
from jax.experimental.pallas import tpu_sc as plsc

VOC = 1000000
DIM = 64
B = 16384
L = 50

NC = 2
NS = 16
NW = NC * NS
B_PER_W = B // NW
NBUF = 4
NLANE = 16
ND = DIM // NLANE
SCALE = 1.0 / L


def _body(items_hbm, table_hbm, out_hbm, idx_v, rows_v, out_v, *sems):
    wid = lax.axis_index("s") * NC + lax.axis_index("c")

    pltpu.sync_copy(items_hbm.at[pl.ds(wid * B_PER_W, B_PER_W)], idx_v)

    def issue(batch, buf):
        return pltpu.async_copy(
            table_hbm.at[idx_v.at[batch]], rows_v.at[buf], sems[buf])

    for b in range(NBUF):
        issue(b, b)

    def reduce_batch(batch, buf):
        def accum(l, accs):
            return tuple(
                accs[d] + rows_v[buf, l, pl.ds(d * NLANE, NLANE)]
                for d in range(ND))

        accs = lax.fori_loop(
            0, L, accum,
            tuple(jnp.zeros((NLANE,), jnp.float32) for _ in range(ND)),
            unroll=2)
        for d in range(ND):
            out_v[batch, pl.ds(d * NLANE, NLANE)] = accs[d] * SCALE

    @pl.loop(0, B_PER_W, step=NBUF)
    def _(c):
        for b in range(NBUF):
            cc = c + b
            pltpu.make_async_copy(
                table_hbm.at[idx_v.at[cc]], rows_v.at[b], sems[b]).wait()
            reduce_batch(cc, b)
            nxt = cc + NBUF

            @pl.when(nxt < B_PER_W)
            def _():
                issue(nxt, b)

    pltpu.sync_copy(out_v, out_hbm.at[pl.ds(wid * B_PER_W, B_PER_W)])


@jax.jit
def _run(items, table):
    mesh = plsc.VectorSubcoreMesh(core_axis_name="c", subcore_axis_name="s")
    return pl.kernel(
        _body,
        out_type=jax.ShapeDtypeStruct((B, DIM), jnp.float32),
        mesh=mesh,
        scratch_types=[
            pltpu.VMEM((B_PER_W, L), jnp.int32),
            pltpu.VMEM((NBUF, L, DIM), jnp.float32),
            pltpu.VMEM((B_PER_W, DIM), jnp.float32),
        ] + [pltpu.SemaphoreType.DMA] * NBUF,
        compiler_params=pltpu.CompilerParams(use_tc_tiling_on_sc=False),
    )(items, table)


def kernel(items, table):
    return _run(items.astype(jnp.int32), table)


# speedup vs baseline: 1.0495x; 1.0495x over previous
import jax
import jax.numpy as jnp
from jax import lax
from jax.experimental import pallas as pl
from jax.experimental.pallas import tpu as pltpu
from jax.experimental.pallas import tpu_sc as plsc

VOC = 1000000
DIM = 64
B = 16384
L = 50

NC = 2
NS = 16
NW = NC * NS
B_PER_W = B // NW
CB = 2
C_PER_W = B_PER_W // CB
NBUF = 4
NLANE = 16
ND = DIM // NLANE
SCALE = 1.0 / L


def _body(items_hbm, table_hbm, out_hbm, idx_v, rows_v, out_v, *sems):
    wid = lax.axis_index("s") * NC + lax.axis_index("c")

    pltpu.sync_copy(items_hbm.at[pl.ds(wid * C_PER_W, C_PER_W)], idx_v)

    def issue(chunk, buf):
        return pltpu.async_copy(
            table_hbm.at[idx_v.at[chunk]], rows_v.at[buf], sems[buf])

    for b in range(NBUF):
        issue(b, b)

    def reduce_chunk(chunk, buf):
        for cb in range(CB):
            def accum(l, accs):
                return tuple(
                    accs[d] + rows_v[buf, cb * L + l, pl.ds(d * NLANE, NLANE)]
                    for d in range(ND))

            accs = lax.fori_loop(
                0, L, accum,
                tuple(jnp.zeros((NLANE,), jnp.float32) for _ in range(ND)),
                unroll=2)
            for d in range(ND):
                out_v[chunk * CB + cb, pl.ds(d * NLANE, NLANE)] = (
                    accs[d] * SCALE)

    @pl.loop(0, C_PER_W, step=NBUF)
    def _(c):
        for b in range(NBUF):
            cc = c + b
            pltpu.make_async_copy(
                table_hbm.at[idx_v.at[cc]], rows_v.at[b], sems[b]).wait()
            reduce_chunk(cc, b)
            nxt = cc + NBUF

            @pl.when(nxt < C_PER_W)
            def _():
                issue(nxt, b)

    pltpu.sync_copy(out_v, out_hbm.at[pl.ds(wid * B_PER_W, B_PER_W)])


@jax.jit
def _run(items, table):
    mesh = plsc.VectorSubcoreMesh(core_axis_name="c", subcore_axis_name="s")
    return pl.kernel(
        _body,
        out_type=jax.ShapeDtypeStruct((B, DIM), jnp.float32),
        mesh=mesh,
        scratch_types=[
            pltpu.VMEM((C_PER_W, CB * L), jnp.int32),
            pltpu.VMEM((NBUF, CB * L, DIM), jnp.float32),
            pltpu.VMEM((B_PER_W, DIM), jnp.float32),
        ] + [pltpu.SemaphoreType.DMA] * NBUF,
        compiler_params=pltpu.CompilerParams(use_tc_tiling_on_sc=False),
    )(items, table)


def kernel(items, table):
    return _run(items.astype(jnp.int32).reshape(B // CB, CB * L), table)


# speedup vs baseline: 1.0747x; 1.0239x over previous
import jax
import jax.numpy as jnp
from jax import lax
from jax.experimental import pallas as pl
from jax.experimental.pallas import tpu as pltpu
from jax.experimental.pallas import tpu_sc as plsc

VOC = 1000000
DIM = 64
B = 16384
L = 50

NC = 2
NS = 16
NW = NC * NS
B_PER_W = B // NW
CB = 2
C_PER_W = B_PER_W // CB
NBUF = 8
NLANE = 16
ND = DIM // NLANE
SCALE = 1.0 / L


def _body(items_hbm, table_hbm, out_hbm, idx_v, rows_v, out_v, *sems):
    wid = lax.axis_index("s") * NC + lax.axis_index("c")

    pltpu.sync_copy(items_hbm.at[pl.ds(wid * C_PER_W, C_PER_W)], idx_v)

    def issue(chunk, buf):
        return pltpu.async_copy(
            table_hbm.at[idx_v.at[chunk]], rows_v.at[buf], sems[buf])

    for b in range(NBUF):
        issue(b, b)

    def reduce_chunk(chunk, buf):
        for cb in range(CB):
            def accum(l, accs):
                return tuple(
                    accs[d] + rows_v[buf, cb * L + l, pl.ds(d * NLANE, NLANE)]
                    for d in range(ND))

            accs = lax.fori_loop(
                0, L, accum,
                tuple(jnp.zeros((NLANE,), jnp.float32) for _ in range(ND)),
                unroll=2)
            for d in range(ND):
                out_v[chunk * CB + cb, pl.ds(d * NLANE, NLANE)] = (
                    accs[d] * SCALE)

    @pl.loop(0, C_PER_W, step=NBUF)
    def _(c):
        for b in range(NBUF):
            cc = c + b
            pltpu.make_async_copy(
                table_hbm.at[idx_v.at[cc]], rows_v.at[b], sems[b]).wait()
            reduce_chunk(cc, b)
            nxt = cc + NBUF

            @pl.when(nxt < C_PER_W)
            def _():
                issue(nxt, b)

    pltpu.sync_copy(out_v, out_hbm.at[pl.ds(wid * B_PER_W, B_PER_W)])


@jax.jit
def _run(items, table):
    mesh = plsc.VectorSubcoreMesh(core_axis_name="c", subcore_axis_name="s")
    return pl.kernel(
        _body,
        out_type=jax.ShapeDtypeStruct((B, DIM), jnp.float32),
        mesh=mesh,
        scratch_types=[
            pltpu.VMEM((C_PER_W, CB * L), jnp.int32),
            pltpu.VMEM((NBUF, CB * L, DIM), jnp.float32),
            pltpu.VMEM((B_PER_W, DIM), jnp.float32),
        ] + [pltpu.SemaphoreType.DMA] * NBUF,
        compiler_params=pltpu.CompilerParams(use_tc_tiling_on_sc=False),
    )(items, table)


def kernel(items, table):
    return _run(items.astype(jnp.int32).reshape(B // CB, CB * L), table)
